# baseline (speedup 1.0000x reference)
import functools

import jax
import jax.numpy as jnp
from jax import lax
from jax.experimental import pallas as pl
from jax.experimental.pallas import tpu as pltpu
from jax.experimental.pallas import tpu_sc as plsc

R = 128
V = 100000
L = 16
NC = 2
NS = 16
NW = NC * NS
RPW = R // NW
NVEC = V // L
U = 10
NIT = NVEC // U

_NEG_INF = float("-inf")


def _gather16(x, idx):
    dnums = lax.GatherDimensionNumbers(
        offset_dims=(), collapsed_slice_dims=(0,), start_index_map=(0,))
    return lax.gather(
        x, idx[:, None], dnums, slice_sizes=(1,),
        mode=lax.GatherScatterMode.PROMISE_IN_BOUNDS)


def _row_argmax(row_v):
    iota = lax.iota(jnp.int32, L)

    def body(i, carry):
        ms, mis, iv = carry
        base = i * (U * L)
        ms_new = []
        mis_new = []
        for j in range(U):
            v = row_v[pl.ds(base + j * L, L)]
            idx = iv + (j * L)
            cmp = v > ms[j]
            ms_new.append(jnp.where(cmp, v, ms[j]))
            mis_new.append(jnp.where(cmp, idx, mis[j]))
        return tuple(ms_new), tuple(mis_new), iv + (U * L)

    ms0 = tuple(jnp.full((L,), _NEG_INF, jnp.float32) for _ in range(U))
    mis0 = tuple(jnp.zeros((L,), jnp.int32) for _ in range(U))
    ms, mis, _ = lax.fori_loop(0, NIT, body, (ms0, mis0, iota))

    m, mi = ms[0], mis[0]
    for j in range(1, U):
        better = (ms[j] > m) | ((ms[j] == m) & (mis[j] < mi))
        m = jnp.where(better, ms[j], m)
        mi = jnp.where(better, mis[j], mi)

    lane = lax.iota(jnp.int32, L)
    for shift in (8, 4, 2, 1):
        perm = (lane + shift) & (L - 1)
        mp = _gather16(m, perm)
        mip = _gather16(mi, perm)
        better = (mp > m) | ((mp == m) & (mip < mi))
        m = jnp.where(better, mp, m)
        mi = jnp.where(better, mip, mi)
    return mi


@functools.partial(
    pl.kernel,
    mesh=plsc.VectorSubcoreMesh(core_axis_name="c", subcore_axis_name="s"),
    out_type=jax.ShapeDtypeStruct((NW, L), jnp.int32),
    scratch_types=[
        pltpu.VMEM((V,), jnp.float32),
        pltpu.VMEM((L,), jnp.int32),
    ],
)
def _argmax_sc(logits_hbm, out_hbm, row_v, out_v):
    cid = lax.axis_index("c")
    sid = lax.axis_index("s")
    wid = sid * NC + cid
    base_row = wid * RPW
    lane = lax.iota(jnp.int32, L)
    res = jnp.zeros((L,), jnp.int32)
    for r in range(RPW):
        pltpu.sync_copy(logits_hbm.at[base_row + r], row_v)
        idx = _row_argmax(row_v)
        res = jnp.where(lane == r, idx, res)
    out_v[...] = res
    pltpu.sync_copy(out_v, out_hbm.at[wid])


def kernel(logits):
    out = _argmax_sc(logits)
    return out[:, :RPW].reshape(R)
